# Initial kernel scaffold; baseline (speedup 1.0000x reference)
import jax
import jax.numpy as jnp
from jax.experimental import pallas as pl


def kernel(pred_boxes, pred_objectness, pred_logits):
    raise NotImplementedError("write your pallas kernel here")


# speedup vs baseline: 162.1004x; 162.1004x over previous
import jax
import jax.numpy as jnp
from jax import lax
from jax.experimental import pallas as pl
from jax.experimental.pallas import tpu as pltpu
from jax.experimental.pallas import tpu_sc as plsc

_B = 4
_N = 5000
_NCLS = 80
_CONF = 0.5
_NMS = 0.5

_L = 16
_NTILE = 16
_NCORE = 2
_IPC = _B // _NCORE
_NP = 6144
_BPT = _NP // _NTILE
_NG = _BPT // _L
_NGI = _NP // _L
_CPT = _NCLS // _NTILE


def _scal(x):
    return x if getattr(x, "ndim", 0) == 0 else x.reshape(-1)[0]


def _nms_body(pb, po, plg,
              ox1, oy1, ox2, oy2, oob, omc,
              hx1, hy1, hx2, hy2, hcf, hcl,
              sbx, sob, slg, stage_cls,
              my_x1, my_y1, my_x2, my_y2, my_conf, my_maxc,
              x1c, y1c, x2c, y2c, cfc, clc,
              morig, alive, keep_copy, keep16, ostg,
              sh_keeps):
    cid = lax.axis_index("c")
    sid = lax.axis_index("s")
    base = sid * _BPT
    iota = lax.broadcasted_iota(jnp.int32, (_L,), 0)
    zeros_i = jnp.zeros((_L,), jnp.int32)
    zeros_f = jnp.zeros((_L,), jnp.float32)

    for li in range(_IPC):
        gimg = cid * _IPC + li
        pltpu.sync_copy(pb.at[gimg].at[pl.ds(base * 4, _BPT * 4)], sbx)
        pltpu.sync_copy(po.at[gimg].at[pl.ds(base, _BPT)], sob)
        pltpu.sync_copy(plg.at[gimg].at[pl.ds(base * _NCLS, _BPT * _NCLS)],
                        slg)

        def p1_body(g, _, li=li):
            s = pl.ds(g * _L, _L)
            rows = iota + g * _L
            rows4 = rows * 4
            cx = plsc.load_gather(sbx, [rows4])
            cy = plsc.load_gather(sbx, [rows4 + 1])
            w = plsc.load_gather(sbx, [rows4 + 2])
            h = plsc.load_gather(sbx, [rows4 + 3])
            obj = sob[s]
            valid = obj > _CONF
            conf = jnp.where(valid, obj, 0.0)

            rowsc = rows * _NCLS
            mv = plsc.load_gather(slg, [rowsc])

            def am_body(cc, st):
                mvv, mii = st
                v = plsc.load_gather(slg, [rowsc + cc])
                b = v > mvv
                return jnp.where(b, v, mvv), jnp.where(b, cc, mii)

            mv, mi = lax.fori_loop(1, _NCLS, am_body, (mv, zeros_i))

            sm = pl.ds(li * _BPT + g * _L, _L)
            my_x1[sm] = cx - w * 0.5
            my_y1[sm] = cy - h * 0.5
            my_x2[sm] = cx + w * 0.5
            my_y2[sm] = cy + h * 0.5
            my_conf[sm] = conf
            my_maxc[sm] = jnp.where(valid, mv, 0.0)
            stage_cls[s] = jnp.where(valid, mi, -1)
            return 0

        lax.fori_loop(0, _NG, p1_body, 0)

        dst = pl.ds(base, _BPT)
        smy = pl.ds(li * _BPT, _BPT)
        pltpu.sync_copy(my_x1.at[smy], hx1.at[gimg].at[dst])
        pltpu.sync_copy(my_y1.at[smy], hy1.at[gimg].at[dst])
        pltpu.sync_copy(my_x2.at[smy], hx2.at[gimg].at[dst])
        pltpu.sync_copy(my_y2.at[smy], hy2.at[gimg].at[dst])
        pltpu.sync_copy(my_conf.at[smy], hcf.at[gimg].at[dst])
        pltpu.sync_copy(stage_cls, hcl.at[gimg].at[dst])

    plsc.subcore_barrier()

    for li in range(_IPC):
        gimg = cid * _IPC + li
        pltpu.sync_copy(hx1.at[gimg], x1c)
        pltpu.sync_copy(hy1.at[gimg], y1c)
        pltpu.sync_copy(hx2.at[gimg], x2c)
        pltpu.sync_copy(hy2.at[gimg], y2c)
        pltpu.sync_copy(hcf.at[gimg], cfc)
        pltpu.sync_copy(hcl.at[gimg], clc)

        def z_body(g, _):
            keep_copy[pl.ds(g * _L, _L)] = zeros_f
            return 0

        lax.fori_loop(0, _NGI, z_body, 0)

        for t in range(_CPT):
            cls_id = sid * _CPT + t

            def scan_body(g, mc, cls_id=cls_id):
                clsv = clc[pl.ds(g * _L, _L)]
                msk = clsv == cls_id
                plsc.store_compressed(morig.at[pl.ds(mc, _L)],
                                      iota + g * _L, mask=msk)
                return mc + _scal(plsc.all_reduce_population_count(msk))

            mcount = lax.fori_loop(0, _NGI, scan_body, jnp.int32(0))
            morig[pl.ds(mcount, _L)] = zeros_i
            nch = (mcount + _L - 1) // _L

            def init_body(j, _):
                s = pl.ds(j * _L, _L)
                pos = iota + j * _L
                alive[s] = jnp.where(pos < mcount, 1.0, 0.0)
                return 0

            lax.fori_loop(0, nch, init_body, 0)

            def nms_body(go):
                def am(j, st):
                    av, ap = st
                    s = pl.ds(j * _L, _L)
                    cv = plsc.load_gather(cfc, [morig[s]])
                    val = jnp.where(alive[s] > 0.0, cv, -1.0)
                    b = val > av
                    return (jnp.where(b, val, av),
                            jnp.where(b, iota + j * _L, ap))

                av, ap = lax.fori_loop(
                    0, nch, am,
                    (jnp.full((_L,), -1.0, jnp.float32), zeros_i))
                bestv = jnp.max(av)
                go2 = bestv > 0.0

                @pl.when(go2)
                def _():
                    cand = jnp.where(av == bestv, ap, jnp.int32(2 ** 30))
                    bp = jnp.min(cand)
                    sb = pl.ds(bp, _L)
                    borig = jnp.full((_L,), morig[sb][0], jnp.int32)
                    bx1 = plsc.load_gather(x1c, [borig])[0]
                    by1 = plsc.load_gather(y1c, [borig])[0]
                    bx2 = plsc.load_gather(x2c, [borig])[0]
                    by2 = plsc.load_gather(y2c, [borig])[0]
                    ba = (bx2 - bx1 + 1.0) * (by2 - by1 + 1.0)
                    plsc.store_scatter(keep_copy, [borig],
                                       jnp.ones((_L,), jnp.float32),
                                       mask=iota == 0)

                    def sweep(j, _):
                        s = pl.ds(j * _L, _L)
                        midx = morig[s]
                        x1v = plsc.load_gather(x1c, [midx])
                        y1v = plsc.load_gather(y1c, [midx])
                        x2v = plsc.load_gather(x2c, [midx])
                        y2v = plsc.load_gather(y2c, [midx])
                        iw = jnp.maximum(
                            jnp.minimum(x2v, bx2) - jnp.maximum(x1v, bx1)
                            + 1.0, 0.0)
                        ih = jnp.maximum(
                            jnp.minimum(y2v, by2) - jnp.maximum(y1v, by1)
                            + 1.0, 0.0)
                        inter = iw * ih
                        areav = (x2v - x1v + 1.0) * (y2v - y1v + 1.0)
                        iou = inter / (areav + ba - inter)
                        alive[s] = jnp.where(iou >= _NMS, 0.0, alive[s])
                        return 0

                    lax.fori_loop(0, nch, sweep, 0)

                return go2

            lax.while_loop(lambda go: go, nms_body, jnp.bool_(True))

        pltpu.sync_copy(keep_copy, sh_keeps.at[li].at[sid])

    plsc.subcore_barrier()

    for li in range(_IPC):
        gimg = cid * _IPC + li
        for r in range(_NTILE):
            pltpu.sync_copy(sh_keeps.at[li].at[r].at[pl.ds(base, _BPT)],
                            keep16.at[pl.ds(r * _BPT, _BPT)])

        def p3_body(g, _, li=li):
            k = keep16[pl.ds(g * _L, _L)]
            for r in range(1, _NTILE):
                k = k + keep16[pl.ds(r * _BPT + g * _L, _L)]
            sm = pl.ds(li * _BPT + g * _L, _L)
            ostg[pl.ds(0 * _BPT + g * _L, _L)] = my_x1[sm] * k
            ostg[pl.ds(1 * _BPT + g * _L, _L)] = my_y1[sm] * k
            ostg[pl.ds(2 * _BPT + g * _L, _L)] = my_x2[sm] * k
            ostg[pl.ds(3 * _BPT + g * _L, _L)] = my_y2[sm] * k
            ostg[pl.ds(4 * _BPT + g * _L, _L)] = my_conf[sm] * k
            ostg[pl.ds(5 * _BPT + g * _L, _L)] = my_maxc[sm] * k
            return 0

        lax.fori_loop(0, _NG, p3_body, 0)
        dst = pl.ds(base, _BPT)
        pltpu.sync_copy(ostg.at[pl.ds(0 * _BPT, _BPT)], ox1.at[gimg].at[dst])
        pltpu.sync_copy(ostg.at[pl.ds(1 * _BPT, _BPT)], oy1.at[gimg].at[dst])
        pltpu.sync_copy(ostg.at[pl.ds(2 * _BPT, _BPT)], ox2.at[gimg].at[dst])
        pltpu.sync_copy(ostg.at[pl.ds(3 * _BPT, _BPT)], oy2.at[gimg].at[dst])
        pltpu.sync_copy(ostg.at[pl.ds(4 * _BPT, _BPT)], oob.at[gimg].at[dst])
        pltpu.sync_copy(ostg.at[pl.ds(5 * _BPT, _BPT)], omc.at[gimg].at[dst])


@jax.jit
def _nms_sc(pb, po, plg):
    f32 = jnp.float32
    i32 = jnp.int32
    out_t = (
        tuple(jax.ShapeDtypeStruct((_B, _NP), f32) for _ in range(6))
        + tuple(jax.ShapeDtypeStruct((_B, _NP), f32) for _ in range(5))
        + (jax.ShapeDtypeStruct((_B, _NP), i32),)
    )
    scratch = [
        pltpu.VMEM((_BPT * 4,), f32),
        pltpu.VMEM((_BPT,), f32),
        pltpu.VMEM((_BPT * _NCLS,), f32),
        pltpu.VMEM((_BPT,), i32),
        pltpu.VMEM((_IPC * _BPT,), f32),
        pltpu.VMEM((_IPC * _BPT,), f32),
        pltpu.VMEM((_IPC * _BPT,), f32),
        pltpu.VMEM((_IPC * _BPT,), f32),
        pltpu.VMEM((_IPC * _BPT,), f32),
        pltpu.VMEM((_IPC * _BPT,), f32),
        pltpu.VMEM((_NP,), f32),
        pltpu.VMEM((_NP,), f32),
        pltpu.VMEM((_NP,), f32),
        pltpu.VMEM((_NP,), f32),
        pltpu.VMEM((_NP,), f32),
        pltpu.VMEM((_NP,), i32),
        pltpu.VMEM((_NP + _L,), i32),
        pltpu.VMEM((_NP + _L,), f32),
        pltpu.VMEM((_NP,), f32),
        pltpu.VMEM((_NTILE * _BPT,), f32),
        pltpu.VMEM((6 * _BPT,), f32),
        pltpu.VMEM_SHARED((_IPC, _NTILE, _NP), f32),
    ]
    mesh = plsc.VectorSubcoreMesh(core_axis_name="c", subcore_axis_name="s")
    run = pl.kernel(_nms_body, out_type=out_t, mesh=mesh,
                    compiler_params=pltpu.CompilerParams(
                        needs_layout_passes=False),
                    scratch_types=scratch)
    return run(pb, po, plg)


def kernel(pred_boxes, pred_objectness, pred_logits):
    pad = _NP - _N
    pb = jnp.pad(pred_boxes, ((0, 0), (0, pad), (0, 0))).reshape(_B, -1)
    po = jnp.pad(pred_objectness[..., 0], ((0, 0), (0, pad)))
    plg = jnp.pad(pred_logits, ((0, 0), (0, pad), (0, 0))).reshape(_B, -1)
    outs = _nms_sc(pb, po, plg)
    x1, y1, x2, y2, ob, mc = outs[:6]
    out = jnp.stack([x1, y1, x2, y2, ob, mc], axis=-1)
    return out[:, :_N, :]


# speedup vs baseline: 177.7799x; 1.0967x over previous
import jax
import jax.numpy as jnp
from jax import lax
from jax.experimental import pallas as pl
from jax.experimental.pallas import tpu as pltpu
from jax.experimental.pallas import tpu_sc as plsc

_B = 4
_N = 5000
_NCLS = 80
_CONF = 0.5
_NMS = 0.5

_L = 16
_NTILE = 16
_NCORE = 2
_IPC = _B // _NCORE
_NP = 6144
_BPT = _NP // _NTILE
_NG = _BPT // _L
_NGI = _NP // _L
_CPT = _NCLS // _NTILE
_NFULL = _N // _BPT
_TAIL = _N - _NFULL * _BPT


def _scal(x):
    return x if getattr(x, "ndim", 0) == 0 else x.reshape(-1)[0]


def _nms_body(pb, po, plg,
              ox1, oy1, ox2, oy2, oob, omc,
              hx1, hy1, hx2, hy2, hcf, hcl,
              sbx, sob, slg, stage_cls,
              my_x1, my_y1, my_x2, my_y2, my_conf, my_maxc,
              x1c, y1c, x2c, y2c, cfc, clc,
              morig, alive, keep_copy, keep16, ostg,
              sh_keeps):
    cid = lax.axis_index("c")
    sid = lax.axis_index("s")
    base = sid * _BPT
    iota = lax.broadcasted_iota(jnp.int32, (_L,), 0)
    zeros_i = jnp.zeros((_L,), jnp.int32)
    zeros_f = jnp.zeros((_L,), jnp.float32)

    for li in range(_IPC):
        gimg = cid * _IPC + li

        pltpu.sync_copy(pb.at[gimg].at[pl.ds(base * 4, _BPT * 4)], sbx)
        pltpu.sync_copy(po.at[gimg].at[pl.ds(base, _BPT)], sob)

        @pl.when(sid < _NFULL)
        def _(li=li, gimg=gimg):
            pltpu.sync_copy(
                plg.at[gimg].at[pl.ds(base * _NCLS, _BPT * _NCLS)], slg)

        @pl.when(sid == _NFULL)
        def _(li=li, gimg=gimg):
            pltpu.sync_copy(plg.at[gimg].at[pl.ds(_NFULL * _BPT * _NCLS,
                                                  _TAIL * _NCLS)],
                            slg.at[pl.ds(0, _TAIL * _NCLS)])

        def p1_body(g, _, li=li):
            s = pl.ds(g * _L, _L)
            rows = iota + g * _L
            rows4 = rows * 4
            cx = plsc.load_gather(sbx, [rows4])
            cy = plsc.load_gather(sbx, [rows4 + 1])
            w = plsc.load_gather(sbx, [rows4 + 2])
            h = plsc.load_gather(sbx, [rows4 + 3])
            obj = sob[s]
            valid = obj > _CONF
            conf = jnp.where(valid, obj, 0.0)

            rowsc = rows * _NCLS
            mv = plsc.load_gather(slg, [rowsc])

            def am_body(cc, st):
                mvv, mii = st
                v = plsc.load_gather(slg, [rowsc + cc])
                b = v > mvv
                return jnp.where(b, v, mvv), jnp.where(b, cc, mii)

            mv, mi = lax.fori_loop(1, _NCLS, am_body, (mv, zeros_i))

            sm = pl.ds(li * _BPT + g * _L, _L)
            my_x1[sm] = cx - w * 0.5
            my_y1[sm] = cy - h * 0.5
            my_x2[sm] = cx + w * 0.5
            my_y2[sm] = cy + h * 0.5
            my_conf[sm] = conf
            my_maxc[sm] = jnp.where(valid, mv, 0.0)
            stage_cls[s] = jnp.where(valid, mi, -1)
            return 0

        lax.fori_loop(0, _NG, p1_body, 0)

        dst = pl.ds(base, _BPT)
        smy = pl.ds(li * _BPT, _BPT)
        pltpu.sync_copy(my_x1.at[smy], hx1.at[gimg].at[dst])
        pltpu.sync_copy(my_y1.at[smy], hy1.at[gimg].at[dst])
        pltpu.sync_copy(my_x2.at[smy], hx2.at[gimg].at[dst])
        pltpu.sync_copy(my_y2.at[smy], hy2.at[gimg].at[dst])
        pltpu.sync_copy(my_conf.at[smy], hcf.at[gimg].at[dst])
        pltpu.sync_copy(stage_cls, hcl.at[gimg].at[dst])

    plsc.subcore_barrier()

    for li in range(_IPC):
        gimg = cid * _IPC + li
        pltpu.sync_copy(hx1.at[gimg], x1c)
        pltpu.sync_copy(hy1.at[gimg], y1c)
        pltpu.sync_copy(hx2.at[gimg], x2c)
        pltpu.sync_copy(hy2.at[gimg], y2c)
        pltpu.sync_copy(hcf.at[gimg], cfc)
        pltpu.sync_copy(hcl.at[gimg], clc)

        def z_body(g, _):
            keep_copy[pl.ds(g * _L, _L)] = zeros_f
            return 0

        lax.fori_loop(0, _NGI, z_body, 0)

        for t in range(_CPT):
            cls_id = sid * _CPT + t

            def scan_body(g, mc, cls_id=cls_id):
                clsv = clc[pl.ds(g * _L, _L)]
                msk = clsv == cls_id
                plsc.store_compressed(morig.at[pl.ds(mc, _L)],
                                      iota + g * _L, mask=msk)
                return mc + _scal(plsc.all_reduce_population_count(msk))

            mcount = lax.fori_loop(0, _NGI, scan_body, jnp.int32(0))
            morig[pl.ds(mcount, _L)] = zeros_i
            nch = (mcount + _L - 1) // _L

            def init_body(j, _):
                s = pl.ds(j * _L, _L)
                pos = iota + j * _L
                alive[s] = jnp.where(pos < mcount, 1.0, 0.0)
                return 0

            lax.fori_loop(0, nch, init_body, 0)

            def am0(j, st):
                av, ap = st
                s = pl.ds(j * _L, _L)
                cv = plsc.load_gather(cfc, [morig[s]])
                val = jnp.where(alive[s] > 0.0, cv, -1.0)
                b = val > av
                return (jnp.where(b, val, av),
                        jnp.where(b, iota + j * _L, ap))

            st0 = lax.fori_loop(
                0, nch, am0, (jnp.full((_L,), -1.0, jnp.float32), zeros_i))

            def nms_cond(st):
                return jnp.max(st[0]) > 0.0

            def nms_body(st):
                av, ap = st
                bestv = jnp.max(av)
                cand = jnp.where(av == bestv, ap, jnp.int32(2 ** 30))
                bp = jnp.min(cand)
                sb = pl.ds(bp, _L)
                borig = jnp.full((_L,), morig[sb][0], jnp.int32)
                bx1 = plsc.load_gather(x1c, [borig])[0]
                by1 = plsc.load_gather(y1c, [borig])[0]
                bx2 = plsc.load_gather(x2c, [borig])[0]
                by2 = plsc.load_gather(y2c, [borig])[0]
                ba = (bx2 - bx1 + 1.0) * (by2 - by1 + 1.0)
                plsc.store_scatter(keep_copy, [borig],
                                   jnp.ones((_L,), jnp.float32),
                                   mask=iota == 0)

                def sweep(j, st2):
                    av2, ap2 = st2
                    s = pl.ds(j * _L, _L)
                    midx = morig[s]
                    x1v = plsc.load_gather(x1c, [midx])
                    y1v = plsc.load_gather(y1c, [midx])
                    x2v = plsc.load_gather(x2c, [midx])
                    y2v = plsc.load_gather(y2c, [midx])
                    iw = jnp.maximum(
                        jnp.minimum(x2v, bx2) - jnp.maximum(x1v, bx1)
                        + 1.0, 0.0)
                    ih = jnp.maximum(
                        jnp.minimum(y2v, by2) - jnp.maximum(y1v, by1)
                        + 1.0, 0.0)
                    inter = iw * ih
                    areav = (x2v - x1v + 1.0) * (y2v - y1v + 1.0)
                    iou = inter / (areav + ba - inter)
                    na = jnp.where(iou >= _NMS, 0.0, alive[s])
                    alive[s] = na
                    cv = plsc.load_gather(cfc, [midx])
                    val = jnp.where(na > 0.0, cv, -1.0)
                    b = val > av2
                    return (jnp.where(b, val, av2),
                            jnp.where(b, iota + j * _L, ap2))

                return lax.fori_loop(
                    0, nch, sweep,
                    (jnp.full((_L,), -1.0, jnp.float32), zeros_i))

            lax.while_loop(nms_cond, nms_body, st0)

        pltpu.sync_copy(keep_copy, sh_keeps.at[li].at[sid])

    plsc.subcore_barrier()

    for li in range(_IPC):
        gimg = cid * _IPC + li
        for r in range(_NTILE):
            pltpu.sync_copy(sh_keeps.at[li].at[r].at[pl.ds(base, _BPT)],
                            keep16.at[pl.ds(r * _BPT, _BPT)])

        def p3_body(g, _, li=li):
            k = keep16[pl.ds(g * _L, _L)]
            for r in range(1, _NTILE):
                k = k + keep16[pl.ds(r * _BPT + g * _L, _L)]
            sm = pl.ds(li * _BPT + g * _L, _L)
            ostg[pl.ds(0 * _BPT + g * _L, _L)] = my_x1[sm] * k
            ostg[pl.ds(1 * _BPT + g * _L, _L)] = my_y1[sm] * k
            ostg[pl.ds(2 * _BPT + g * _L, _L)] = my_x2[sm] * k
            ostg[pl.ds(3 * _BPT + g * _L, _L)] = my_y2[sm] * k
            ostg[pl.ds(4 * _BPT + g * _L, _L)] = my_conf[sm] * k
            ostg[pl.ds(5 * _BPT + g * _L, _L)] = my_maxc[sm] * k
            return 0

        lax.fori_loop(0, _NG, p3_body, 0)
        dst = pl.ds(base, _BPT)
        pltpu.sync_copy(ostg.at[pl.ds(0 * _BPT, _BPT)], ox1.at[gimg].at[dst])
        pltpu.sync_copy(ostg.at[pl.ds(1 * _BPT, _BPT)], oy1.at[gimg].at[dst])
        pltpu.sync_copy(ostg.at[pl.ds(2 * _BPT, _BPT)], ox2.at[gimg].at[dst])
        pltpu.sync_copy(ostg.at[pl.ds(3 * _BPT, _BPT)], oy2.at[gimg].at[dst])
        pltpu.sync_copy(ostg.at[pl.ds(4 * _BPT, _BPT)], oob.at[gimg].at[dst])
        pltpu.sync_copy(ostg.at[pl.ds(5 * _BPT, _BPT)], omc.at[gimg].at[dst])


@jax.jit
def _nms_sc(pb, po, plg):
    f32 = jnp.float32
    i32 = jnp.int32
    out_t = (
        tuple(jax.ShapeDtypeStruct((_B, _NP), f32) for _ in range(6))
        + tuple(jax.ShapeDtypeStruct((_B, _NP), f32) for _ in range(5))
        + (jax.ShapeDtypeStruct((_B, _NP), i32),)
    )
    scratch = [
        pltpu.VMEM((_BPT * 4,), f32),
        pltpu.VMEM((_BPT,), f32),
        pltpu.VMEM((_BPT * _NCLS,), f32),
        pltpu.VMEM((_BPT,), i32),
        pltpu.VMEM((_IPC * _BPT,), f32),
        pltpu.VMEM((_IPC * _BPT,), f32),
        pltpu.VMEM((_IPC * _BPT,), f32),
        pltpu.VMEM((_IPC * _BPT,), f32),
        pltpu.VMEM((_IPC * _BPT,), f32),
        pltpu.VMEM((_IPC * _BPT,), f32),
        pltpu.VMEM((_NP,), f32),
        pltpu.VMEM((_NP,), f32),
        pltpu.VMEM((_NP,), f32),
        pltpu.VMEM((_NP,), f32),
        pltpu.VMEM((_NP,), f32),
        pltpu.VMEM((_NP,), i32),
        pltpu.VMEM((_NP + _L,), i32),
        pltpu.VMEM((_NP + _L,), f32),
        pltpu.VMEM((_NP,), f32),
        pltpu.VMEM((_NTILE * _BPT,), f32),
        pltpu.VMEM((6 * _BPT,), f32),
        pltpu.VMEM_SHARED((_IPC, _NTILE, _NP), f32),
    ]
    mesh = plsc.VectorSubcoreMesh(core_axis_name="c", subcore_axis_name="s")
    run = pl.kernel(_nms_body, out_type=out_t, mesh=mesh,
                    compiler_params=pltpu.CompilerParams(
                        needs_layout_passes=False),
                    scratch_types=scratch)
    return run(pb, po, plg)


def kernel(pred_boxes, pred_objectness, pred_logits):
    pad = _NP - _N
    pb = jnp.pad(pred_boxes, ((0, 0), (0, pad), (0, 0))).reshape(_B, -1)
    po = jnp.pad(pred_objectness[..., 0], ((0, 0), (0, pad)))
    plg = pred_logits.reshape(_B, -1)
    outs = _nms_sc(pb, po, plg)
    x1, y1, x2, y2, ob, mc = outs[:6]
    out = jnp.stack([x1, y1, x2, y2, ob, mc], axis=-1)
    return out[:, :_N, :]
